# baseline (speedup 1.0000x reference)
import jax
import jax.numpy as jnp
from jax import lax
from jax.experimental import pallas as pl
from jax.experimental.pallas import tpu as pltpu
from jax.experimental.pallas import tpu_sc as plsc

_INFO = plsc.get_sparse_core_info()
_NC, _NS, _L = _INFO.num_cores, _INFO.num_subcores, _INFO.num_lanes
_NW = _NC * _NS
_B = 16384
_K = 16
_BPW = _B // _NW
_CH = 128
_NCH = _BPW // _CH
_BLK = _BPW // _L


def _fm_body(user_hbm, product_hbm, ue_hbm, pe_hbm, ub_hbm, pb_hbm, bias_hbm,
             out_hbm, idx_u, idx_p, rows_u, rows_p, ub_v, pb_v, bias_v, out_v,
             sem):
    wid = lax.axis_index("s") * _NC + lax.axis_index("c")
    base = wid * _BPW

    for j in range(_NCH):
        pltpu.sync_copy(user_hbm.at[pl.ds(base + j * _CH, _CH)], idx_u.at[j])
        pltpu.sync_copy(product_hbm.at[pl.ds(base + j * _CH, _CH)], idx_p.at[j])
    pltpu.sync_copy(bias_hbm, bias_v)

    copies = []
    for j in range(_NCH):
        sl = pl.ds(j * _CH, _CH)
        copies.append(pltpu.async_copy(ue_hbm.at[idx_u.at[j]], rows_u.at[sl], sem))
        copies.append(pltpu.async_copy(pe_hbm.at[idx_p.at[j]], rows_p.at[sl], sem))
        copies.append(pltpu.async_copy(ub_hbm.at[idx_u.at[j]], ub_v.at[sl], sem))
        copies.append(pltpu.async_copy(pb_hbm.at[idx_p.at[j]], pb_v.at[sl], sem))
    for c in copies:
        c.wait()

    bias_vec = bias_v[...]
    iota = lax.iota(jnp.int32, _L)

    def block(b, carry):
        g = b * _L + iota
        acc = bias_vec + ub_v[pl.ds(b * _L, _L)] + pb_v[pl.ds(b * _L, _L)]
        for k in range(_K):
            kv = jnp.full((_L,), k, jnp.int32)
            acc = acc + (plsc.load_gather(rows_u, [g, kv])
                         * plsc.load_gather(rows_p, [g, kv]))
        out_v[pl.ds(b * _L, _L)] = acc
        return carry

    lax.fori_loop(0, _BLK, block, 0)
    pltpu.sync_copy(out_v, out_hbm.at[pl.ds(base, _BPW)])


@jax.jit
def kernel(user, product, user_embeds, products_embeds, user_bias,
           products_bias, bias):
    bias16 = jnp.broadcast_to(bias, (_L,))
    f = pl.kernel(
        _fm_body,
        out_type=jax.ShapeDtypeStruct((_B,), jnp.float32),
        mesh=plsc.VectorSubcoreMesh(core_axis_name="c", subcore_axis_name="s"),
        compiler_params=pltpu.CompilerParams(use_tc_tiling_on_sc=False,
                                             needs_layout_passes=False),
        scratch_types=[
            pltpu.VMEM((_NCH, _CH), jnp.int32),
            pltpu.VMEM((_NCH, _CH), jnp.int32),
            pltpu.VMEM((_BPW, _K), jnp.float32),
            pltpu.VMEM((_BPW, _K), jnp.float32),
            pltpu.VMEM((_BPW,), jnp.float32),
            pltpu.VMEM((_BPW,), jnp.float32),
            pltpu.VMEM((_L,), jnp.float32),
            pltpu.VMEM((_BPW,), jnp.float32),
            pltpu.SemaphoreType.DMA,
        ],
    )
    return f(user, product, user_embeds, products_embeds,
             user_bias.reshape(-1), products_bias.reshape(-1), bias16)
